# Initial kernel scaffold; baseline (speedup 1.0000x reference)
import jax
import jax.numpy as jnp
from jax.experimental import pallas as pl


def kernel(t, Wp, bp, Wf, bf):
    raise NotImplementedError("write your pallas kernel here")


# speedup vs baseline: 1.0921x; 1.0921x over previous
import jax
import jax.numpy as jnp
from jax.experimental import pallas as pl
from jax.experimental.pallas import tpu as pltpu

TILE_V = 2048


def _proj_gelu_body(t_ref, wp_ref, bp_ref, x_ref):
    proj = jax.lax.dot_general(
        t_ref[...], wp_ref[...],
        (((1,), (1,)), ((), ())),
        preferred_element_type=jnp.float32)
    proj = proj + bp_ref[...]
    g = 0.5 * proj * (1.0 + jax.lax.erf(proj * 0.7071067811865476))
    x_ref[...] = g.astype(jnp.bfloat16)


def _fc_body(x_ref, wf_ref, bf_ref, out_ref):
    wf = wf_ref[...].astype(jnp.bfloat16)
    acc = jax.lax.dot_general(
        x_ref[...], wf,
        (((1,), (1,)), ((), ())),
        preferred_element_type=jnp.float32)
    out_ref[...] = acc + bf_ref[...]


def kernel(t, Wp, bp, Wf, bf):
    B, S, H = t.shape
    P, _ = Wp.shape
    V, _ = Wf.shape
    M = B * S
    t2 = t.reshape(M, H)

    x = pl.pallas_call(
        _proj_gelu_body,
        out_shape=jax.ShapeDtypeStruct((M, P), jnp.bfloat16),
    )(t2, Wp, bp.reshape(1, P))

    out = pl.pallas_call(
        _fc_body,
        grid=(pl.cdiv(V, TILE_V),),
        in_specs=[
            pl.BlockSpec((M, P), lambda v: (0, 0)),
            pl.BlockSpec((TILE_V, P), lambda v: (v, 0)),
            pl.BlockSpec((1, TILE_V), lambda v: (0, v)),
        ],
        out_specs=pl.BlockSpec((M, TILE_V), lambda v: (0, v)),
        out_shape=jax.ShapeDtypeStruct((M, V), jnp.float32),
        compiler_params=pltpu.CompilerParams(
            dimension_semantics=("parallel",)),
    )(x, Wf, bf.reshape(1, V))
    return out.reshape(B, S, V)


# speedup vs baseline: 1.1297x; 1.0345x over previous
import jax
import jax.numpy as jnp
from jax.experimental import pallas as pl
from jax.experimental.pallas import tpu as pltpu

TILE_V = 4096


def _proj_gelu_body(t_ref, wp_ref, bp_ref, x_ref):
    proj = jax.lax.dot_general(
        t_ref[...], wp_ref[...],
        (((1,), (1,)), ((), ())),
        preferred_element_type=jnp.float32)
    proj = proj + bp_ref[...]
    g = 0.5 * proj * (1.0 + jax.lax.erf(proj * 0.7071067811865476))
    x_ref[...] = g.astype(jnp.bfloat16)


def _fc_body(x_ref, wf_ref, bf_ref, out_ref):
    wf = wf_ref[...].astype(jnp.bfloat16)
    acc = jax.lax.dot_general(
        x_ref[...], wf,
        (((1,), (1,)), ((), ())),
        preferred_element_type=jnp.float32)
    out_ref[...] = acc + bf_ref[...]


def kernel(t, Wp, bp, Wf, bf):
    B, S, H = t.shape
    P, _ = Wp.shape
    V, _ = Wf.shape
    M = B * S
    t2 = t.reshape(M, H)

    x = pl.pallas_call(
        _proj_gelu_body,
        out_shape=jax.ShapeDtypeStruct((M, P), jnp.bfloat16),
    )(t2, Wp, bp.reshape(1, P))

    out = pl.pallas_call(
        _fc_body,
        grid=(pl.cdiv(V, TILE_V),),
        in_specs=[
            pl.BlockSpec((M, P), lambda v: (0, 0)),
            pl.BlockSpec((TILE_V, P), lambda v: (v, 0)),
            pl.BlockSpec((1, TILE_V), lambda v: (0, v)),
        ],
        out_specs=pl.BlockSpec((M, TILE_V), lambda v: (0, v)),
        out_shape=jax.ShapeDtypeStruct((M, V), jnp.float32),
        compiler_params=pltpu.CompilerParams(
            dimension_semantics=("parallel",)),
    )(x, Wf, bf.reshape(1, V))
    return out.reshape(B, S, V)
